# Initial kernel scaffold; baseline (speedup 1.0000x reference)
import functools
import jax
import jax.numpy as jnp
from jax import lax
from jax.experimental import pallas as pl
from jax.experimental.pallas import tpu as pltpu
from jax.experimental.pallas import tpu_sc as plsc

NC, NS = 2, 16
NW = NC * NS
CH = 100
D = 128
EW = 8
BK = 1000


def _sc_gather(ent_embed, rel_embed, src2d, dst2d, rel2d, n):
    chunks = n // CH
    per_w = chunks // NW
    mesh = plsc.VectorSubcoreMesh(core_axis_name="c", subcore_axis_name="s")

    @functools.partial(
        pl.kernel, mesh=mesh,
        out_type=(jax.ShapeDtypeStruct((n, D), jnp.float32),) * 3,
        scratch_types=[
            pltpu.VMEM((CH,), jnp.int32),
            pltpu.VMEM((CH, D), jnp.float32),
            pltpu.SemaphoreType.DMA,
        ],
    )
    def k(ent_hbm, rel_hbm, src_hbm, dst_hbm, relid_hbm,
          u_hbm, v_hbm, r_hbm, idx_v, rows_v, sem):
        wid = lax.axis_index("s") * NC + lax.axis_index("c")

        def body(i, carry):
            ck = wid * per_w + i
            row0 = ck * CH
            pltpu.sync_copy(src_hbm.at[ck], idx_v)
            pltpu.async_copy(ent_hbm.at[idx_v], rows_v, sem).wait()
            pltpu.sync_copy(rows_v, u_hbm.at[pl.ds(row0, CH)])
            pltpu.sync_copy(dst_hbm.at[ck], idx_v)
            pltpu.async_copy(ent_hbm.at[idx_v], rows_v, sem).wait()
            pltpu.sync_copy(rows_v, v_hbm.at[pl.ds(row0, CH)])
            pltpu.sync_copy(relid_hbm.at[ck], idx_v)
            pltpu.async_copy(rel_hbm.at[idx_v], rows_v, sem).wait()
            pltpu.sync_copy(rows_v, r_hbm.at[pl.ds(row0, CH)])
            return carry

        lax.fori_loop(0, per_w, body, 0)

    return k(ent_embed, rel_embed, src2d, dst2d, rel2d)


def _sc_scatter(fc, bc, fe, be, src2d, dst2d, rel2d, zc, ze, n, n_seg):
    rows_pt = n_seg // NS
    chunks = n // CH
    per_tile = chunks // NS
    mesh = plsc.VectorSubcoreMesh(core_axis_name="c", subcore_axis_name="s")

    @functools.partial(
        pl.kernel, mesh=mesh,
        out_type=(jax.ShapeDtypeStruct((n_seg, D), jnp.float32),
                  jax.ShapeDtypeStruct((n_seg, EW), jnp.float32),
                  jax.ShapeDtypeStruct((n_seg, D), jnp.float32),
                  jax.ShapeDtypeStruct((n_seg, EW), jnp.float32)),
        scratch_types=[
            pltpu.VMEM_SHARED((n_seg, D), jnp.float32),
            pltpu.VMEM_SHARED((n_seg, EW), jnp.float32),
            pltpu.VMEM((CH,), jnp.int32),
            pltpu.VMEM((CH, D), jnp.float32),
            pltpu.VMEM((CH, EW), jnp.float32),
            pltpu.VMEM((n_seg // NS, D), jnp.float32),
            pltpu.VMEM((n_seg // NS, EW), jnp.float32),
        ],
    )
    def k(fc_hbm, bc_hbm, fe_hbm, be_hbm, src_hbm, dst_hbm, rel_hbm,
          zc_hbm, ze_hbm,
          entc_hbm, ente_hbm, relc_hbm, rele_hbm,
          acc_c, acc_e, idx_v, rows_v, e_v, big_c, big_e):
        cid = lax.axis_index("c")
        sid = lax.axis_index("s")
        r0 = sid * rows_pt

        pltpu.sync_copy(zc_hbm, big_c)
        pltpu.sync_copy(big_c, acc_c.at[pl.ds(r0, rows_pt)])
        pltpu.sync_copy(ze_hbm, big_e)
        pltpu.sync_copy(big_e, acc_e.at[pl.ds(r0, rows_pt)])
        plsc.subcore_barrier()

        def scan_half(data_c, data_e, ids2d):
            def body(i, carry):
                ck = sid * per_tile + i
                row0 = ck * CH
                pltpu.sync_copy(ids2d.at[ck], idx_v)
                pltpu.sync_copy(data_c.at[pl.ds(row0, CH)], rows_v)
                pltpu.sync_copy(data_e.at[pl.ds(row0, CH)], e_v)
                pltpu.sync_copy(rows_v, acc_c.at[idx_v], add=True)
                pltpu.sync_copy(e_v, acc_e.at[idx_v], add=True)
                return carry
            lax.fori_loop(0, per_tile, body, 0)

        @pl.when(cid == 0)
        def _():
            scan_half(fc_hbm, fe_hbm, src_hbm)
            scan_half(bc_hbm, be_hbm, dst_hbm)

        @pl.when(cid == 1)
        def _():
            scan_half(fc_hbm, fe_hbm, rel_hbm)

        plsc.subcore_barrier()

        pltpu.sync_copy(acc_c.at[pl.ds(r0, rows_pt)], big_c)
        pltpu.sync_copy(acc_e.at[pl.ds(r0, rows_pt)], big_e)

        @pl.when(cid == 0)
        def _():
            pltpu.sync_copy(big_c, entc_hbm.at[pl.ds(r0, rows_pt)])
            pltpu.sync_copy(big_e, ente_hbm.at[pl.ds(r0, rows_pt)])

        @pl.when(cid == 1)
        def _():
            pltpu.sync_copy(big_c, relc_hbm.at[pl.ds(r0, rows_pt)])
            pltpu.sync_copy(big_e, rele_hbm.at[pl.ds(r0, rows_pt)])

    return k(fc, bc, fe, be, src2d, dst2d, rel2d, zc, ze)


def _tc_stats(u, v, r):
    n = u.shape[0]
    grid = n // BK

    def body(u_ref, v_ref, r_ref, o_ref):
        @pl.when(pl.program_id(0) == 0)
        def _():
            o_ref[...] = jnp.zeros_like(o_ref)
        for j, x_ref in enumerate((u_ref, v_ref, r_ref)):
            x = x_ref[...]
            o_ref[pl.ds(2 * j, 1), :] += jnp.sum(x, 0, keepdims=True)
            o_ref[pl.ds(2 * j + 1, 1), :] += jnp.sum(x * x, 0, keepdims=True)

    blk = pl.BlockSpec((BK, D), lambda i: (i, 0))
    return pl.pallas_call(
        body,
        grid=(grid,),
        in_specs=[blk, blk, blk],
        out_specs=pl.BlockSpec((8, D), lambda i: (0, 0)),
        out_shape=jax.ShapeDtypeStruct((8, D), jnp.float32),
    )(u, v, r)


def _tc_matmul(u, v, r, w0, w1, w2, bp):
    n = u.shape[0]
    grid = n // BK

    def body(u_ref, v_ref, r_ref, w0_ref, w1_ref, w2_ref, bp_ref,
             cf_ref, cb_ref, st_ref):
        uu = u_ref[...]
        vv = v_ref[...]
        rr = r_ref[...]
        w0m = w0_ref[...]
        w1m = w1_ref[...]
        s = jnp.dot(rr, w2_ref[...], preferred_element_type=jnp.float32)
        cf = (jnp.dot(uu, w0m, preferred_element_type=jnp.float32)
              + jnp.dot(vv, w1m, preferred_element_type=jnp.float32)
              + s + bp_ref[...])
        cb = (jnp.dot(vv, w0m, preferred_element_type=jnp.float32)
              + jnp.dot(uu, w1m, preferred_element_type=jnp.float32)
              - s + bp_ref[...])
        cf_ref[...] = cf
        cb_ref[...] = cb

        @pl.when(pl.program_id(0) == 0)
        def _():
            st_ref[...] = jnp.zeros_like(st_ref)
        st_ref[pl.ds(0, 1), :] += (jnp.sum(cf, 0, keepdims=True)
                                   + jnp.sum(cb, 0, keepdims=True))
        st_ref[pl.ds(1, 1), :] += (jnp.sum(cf * cf, 0, keepdims=True)
                                   + jnp.sum(cb * cb, 0, keepdims=True))

    blk = pl.BlockSpec((BK, D), lambda i: (i, 0))
    wblk = pl.BlockSpec((D, D), lambda i: (0, 0))
    return pl.pallas_call(
        body,
        grid=(grid,),
        in_specs=[blk, blk, blk, wblk, wblk, wblk,
                  pl.BlockSpec((1, D), lambda i: (0, 0))],
        out_specs=[blk, blk, pl.BlockSpec((8, D), lambda i: (0, 0))],
        out_shape=[jax.ShapeDtypeStruct((n, D), jnp.float32),
                   jax.ShapeDtypeStruct((n, D), jnp.float32),
                   jax.ShapeDtypeStruct((8, D), jnp.float32)],
    )(u, v, r, w0, w1, w2, bp)


def _tc_edge(cf, cb, par):
    n = cf.shape[0]
    grid = n // BK

    def half(c_ref, par_ref, oc_ref, oe_ref):
        c = c_ref[...]
        alpha = par_ref[pl.ds(0, 1), :]
        delta = par_ref[pl.ds(1, 1), :]
        w2v = par_ref[pl.ds(2, 1), :]
        b2 = par_ref[pl.ds(3, 1), pl.ds(0, 1)]
        a = jax.lax.dot_general(c, w2v,
                                (((1,), (1,)), ((), ())),
                                preferred_element_type=jnp.float32)
        a = a + b2
        e = jnp.exp(-jnp.where(a > 0, a, 0.01 * a))
        c2 = c * alpha + delta
        oc_ref[...] = e * c2
        col = lax.broadcasted_iota(jnp.int32, (BK, EW), 1)
        ones = jnp.ones((BK, EW), jnp.float32)
        oe_ref[...] = jnp.where(col == 0, e, jnp.where(col == 1, ones, 0.0))

    def body(cf_ref, cb_ref, par_ref, fc_ref, bc_ref, fe_ref, be_ref):
        half(cf_ref, par_ref, fc_ref, fe_ref)
        half(cb_ref, par_ref, bc_ref, be_ref)

    blk = pl.BlockSpec((BK, D), lambda i: (i, 0))
    eblk = pl.BlockSpec((BK, EW), lambda i: (i, 0))
    return pl.pallas_call(
        body,
        grid=(grid,),
        in_specs=[blk, blk, pl.BlockSpec((8, D), lambda i: (0, 0))],
        out_specs=[blk, blk, eblk, eblk],
        out_shape=[jax.ShapeDtypeStruct((n, D), jnp.float32),
                   jax.ShapeDtypeStruct((n, D), jnp.float32),
                   jax.ShapeDtypeStruct((n, EW), jnp.float32),
                   jax.ShapeDtypeStruct((n, EW), jnp.float32)],
    )(cf, cb, par)


def _tc_divide(entc, ente, relc, rele):
    n_seg = entc.shape[0]
    grid = n_seg // BK

    def body(ec_ref, ee_ref, rc_ref, re_ref, he_ref, hr_ref):
        ebs = ee_ref[:, pl.ds(0, 1)]
        he_ref[...] = ec_ref[...] / jnp.where(ebs == 0.0, 1e-12, ebs)
        cnt = re_ref[:, pl.ds(1, 1)]
        hr_ref[...] = rc_ref[...] / jnp.maximum(cnt, 1.0)

    blk = pl.BlockSpec((BK, D), lambda i: (i, 0))
    eblk = pl.BlockSpec((BK, EW), lambda i: (i, 0))
    return pl.pallas_call(
        body,
        grid=(grid,),
        in_specs=[blk, eblk, blk, eblk],
        out_specs=[blk, blk],
        out_shape=[jax.ShapeDtypeStruct((n_seg, D), jnp.float32),
                   jax.ShapeDtypeStruct((n_seg, D), jnp.float32)],
    )(entc, ente, relc, rele)


@jax.jit
def kernel(triplets, ent_embed, rel_embed, W_a, b_a, W_a2, b_a2,
           gamma0, beta0, gamma1, beta1):
    n = triplets.shape[0]
    n_seg = ent_embed.shape[0]
    N = jnp.float32(2 * n)
    eps = jnp.float32(1e-5)

    src2d = triplets[:, 0].reshape(n // CH, CH)
    dst2d = triplets[:, 1].reshape(n // CH, CH)
    rel2d = triplets[:, 2].reshape(n // CH, CH)

    u, v, r = _sc_gather(ent_embed, rel_embed, src2d, dst2d, rel2d, n)

    st = _tc_stats(u, v, r)
    s01 = st[0] + st[2]
    q01 = st[1] + st[3]
    m01 = s01 / N
    var01 = q01 / N - m01 * m01
    var2 = 2.0 * st[5] / N
    m = jnp.concatenate([m01, m01, jnp.zeros_like(m01)])
    var = jnp.concatenate([var01, var01, var2])
    sfold = gamma0 * jax.lax.rsqrt(var + eps)
    Wp = W_a * sfold[None, :]
    bp = (b_a + W_a @ (beta0 - m * sfold)).reshape(1, D)
    w0 = Wp[:, 0:D].T
    w1 = Wp[:, D:2 * D].T
    w2 = Wp[:, 2 * D:3 * D].T

    cf, cb, cst = _tc_matmul(u, v, r, w0, w1, w2, bp)
    m2 = cst[0] / N
    v2 = cst[1] / N - m2 * m2
    alpha = gamma1 * jax.lax.rsqrt(v2 + eps)
    delta = beta1 - m2 * alpha
    w2v = W_a2[0] * alpha
    b2 = b_a2[0] + W_a2[0] @ delta
    par = jnp.zeros((8, D), jnp.float32)
    par = par.at[0].set(alpha).at[1].set(delta).at[2].set(w2v)
    par = par.at[3, 0].set(b2)

    fc, bc, fe, be = _tc_edge(cf, cb, par)

    zc = jnp.zeros((n_seg // NS, D), jnp.float32)
    ze = jnp.zeros((n_seg // NS, EW), jnp.float32)
    entc, ente, relc, rele = _sc_scatter(fc, bc, fe, be,
                                         src2d, dst2d, rel2d, zc, ze,
                                         n, n_seg)

    h_ent, h_rel = _tc_divide(entc, ente, relc, rele)
    return (h_ent, h_rel)


# speedup vs baseline: 1.6347x; 1.6347x over previous
import functools
import jax
import jax.numpy as jnp
from jax import lax
from jax.experimental import pallas as pl
from jax.experimental.pallas import tpu as pltpu
from jax.experimental.pallas import tpu_sc as plsc

NC, NS = 2, 16
NW = NC * NS
CH = 40
D = 128
BK = 1000
GR = 624
WB = 48


def _sc_gather(ent_embed, rel_embed, src, dst, rel, n):
    per_w = n // (CH * NW)
    mesh = plsc.VectorSubcoreMesh(core_axis_name="c", subcore_axis_name="s")

    @functools.partial(
        pl.kernel, mesh=mesh,
        out_type=(jax.ShapeDtypeStruct((n, D), jnp.float32),) * 3,
        scratch_types=[
            pltpu.VMEM((CH,), jnp.int32),
            pltpu.VMEM((CH, D), jnp.float32),
            pltpu.SemaphoreType.DMA,
        ],
    )
    def k(ent_hbm, rel_hbm, src_hbm, dst_hbm, relid_hbm,
          u_hbm, v_hbm, r_hbm, idx_v, rows_v, sem):
        wid = lax.axis_index("s") * NC + lax.axis_index("c")

        def body(i, carry):
            row0 = (wid * per_w + i) * CH
            pltpu.sync_copy(src_hbm.at[pl.ds(row0, CH)], idx_v)
            pltpu.async_copy(ent_hbm.at[idx_v], rows_v, sem).wait()
            pltpu.sync_copy(rows_v, u_hbm.at[pl.ds(row0, CH)])
            pltpu.sync_copy(dst_hbm.at[pl.ds(row0, CH)], idx_v)
            pltpu.async_copy(ent_hbm.at[idx_v], rows_v, sem).wait()
            pltpu.sync_copy(rows_v, v_hbm.at[pl.ds(row0, CH)])
            pltpu.sync_copy(relid_hbm.at[pl.ds(row0, CH)], idx_v)
            pltpu.async_copy(rel_hbm.at[idx_v], rows_v, sem).wait()
            pltpu.sync_copy(rows_v, r_hbm.at[pl.ds(row0, CH)])
            return carry

        lax.fori_loop(0, per_w, body, 0)

    return k(ent_embed, rel_embed, src, dst, rel)


def _sc_scatter(fc, bc, fe, be, src, dst, rel, zc, zs, ones_in, n, n_seg):
    per_tile = n // (CH * NS)
    tail0 = NS * GR
    tail = n_seg - tail0
    mesh = plsc.VectorSubcoreMesh(core_axis_name="c", subcore_axis_name="s")

    @functools.partial(
        pl.kernel, mesh=mesh,
        out_type=(jax.ShapeDtypeStruct((n_seg, D), jnp.float32),
                  jax.ShapeDtypeStruct((n_seg,), jnp.float32),
                  jax.ShapeDtypeStruct((n_seg, D), jnp.float32),
                  jax.ShapeDtypeStruct((n_seg,), jnp.float32)),
        scratch_types=[
            pltpu.VMEM_SHARED((n_seg, D), jnp.float32),
            pltpu.VMEM_SHARED((n_seg,), jnp.float32),
            pltpu.VMEM((CH,), jnp.int32),
            pltpu.VMEM((CH, D), jnp.float32),
            pltpu.VMEM((CH,), jnp.float32),
            pltpu.VMEM((CH,), jnp.float32),
            pltpu.VMEM((WB, D), jnp.float32),
            pltpu.VMEM((GR,), jnp.float32),
            pltpu.VMEM((16, D), jnp.float32),
            pltpu.VMEM((16,), jnp.float32),
        ],
    )
    def k(fc_hbm, bc_hbm, fe_hbm, be_hbm, src_hbm, dst_hbm, rel_hbm,
          zc_hbm, zs_hbm, ones_hbm,
          entc_hbm, ents_hbm, relc_hbm, rels_hbm,
          acc_c, acc_s, idx_v, rows_v, e_v, ones_v, big_c, srow_v,
          sm_c, sm_s):
        cid = lax.axis_index("c")
        sid = lax.axis_index("s")
        r0 = sid * GR

        pltpu.sync_copy(zc_hbm, big_c)
        for j in range(GR // WB):
            pltpu.sync_copy(big_c, acc_c.at[pl.ds(r0 + j * WB, WB)])
        pltpu.sync_copy(zs_hbm, srow_v)
        pltpu.sync_copy(srow_v, acc_s.at[pl.ds(r0, GR)])
        pltpu.sync_copy(ones_hbm, ones_v)

        @pl.when(sid == 0)
        def _():
            pltpu.sync_copy(zc_hbm.at[pl.ds(0, tail)], sm_c)
            pltpu.sync_copy(sm_c, acc_c.at[pl.ds(tail0, tail)])
            pltpu.sync_copy(zs_hbm.at[pl.ds(0, tail)], sm_s)
            pltpu.sync_copy(sm_s, acc_s.at[pl.ds(tail0, tail)])

        plsc.subcore_barrier()

        def scan_ent(data_c, data_e, ids):
            def body(i, carry):
                row0 = (sid * per_tile + i) * CH
                pltpu.sync_copy(ids.at[pl.ds(row0, CH)], idx_v)
                pltpu.sync_copy(data_c.at[pl.ds(row0, CH)], rows_v)
                pltpu.sync_copy(data_e.at[pl.ds(row0, CH)], e_v)
                pltpu.sync_copy(rows_v, acc_c.at[idx_v], add=True)
                pltpu.sync_copy(e_v, acc_s.at[idx_v], add=True)
                return carry
            lax.fori_loop(0, per_tile, body, 0)

        def scan_rel(data_c, ids):
            def body(i, carry):
                row0 = (sid * per_tile + i) * CH
                pltpu.sync_copy(ids.at[pl.ds(row0, CH)], idx_v)
                pltpu.sync_copy(data_c.at[pl.ds(row0, CH)], rows_v)
                pltpu.sync_copy(rows_v, acc_c.at[idx_v], add=True)
                pltpu.sync_copy(ones_v, acc_s.at[idx_v], add=True)
                return carry
            lax.fori_loop(0, per_tile, body, 0)

        @pl.when(cid == 0)
        def _():
            scan_ent(fc_hbm, fe_hbm, src_hbm)
            scan_ent(bc_hbm, be_hbm, dst_hbm)

        @pl.when(cid == 1)
        def _():
            scan_rel(fc_hbm, rel_hbm)

        plsc.subcore_barrier()

        for j in range(GR // WB):
            rj = r0 + j * WB
            pltpu.sync_copy(acc_c.at[pl.ds(rj, WB)], big_c)

            @pl.when(cid == 0)
            def _():
                pltpu.sync_copy(big_c, entc_hbm.at[pl.ds(rj, WB)])

            @pl.when(cid == 1)
            def _():
                pltpu.sync_copy(big_c, relc_hbm.at[pl.ds(rj, WB)])

        pltpu.sync_copy(acc_s.at[pl.ds(r0, GR)], srow_v)

        @pl.when(cid == 0)
        def _():
            pltpu.sync_copy(srow_v, ents_hbm.at[pl.ds(r0, GR)])

        @pl.when(cid == 1)
        def _():
            pltpu.sync_copy(srow_v, rels_hbm.at[pl.ds(r0, GR)])

        @pl.when(sid == 0)
        def _():
            pltpu.sync_copy(acc_c.at[pl.ds(tail0, tail)], sm_c)
            pltpu.sync_copy(acc_s.at[pl.ds(tail0, tail)], sm_s)

            @pl.when(cid == 0)
            def _():
                pltpu.sync_copy(sm_c, entc_hbm.at[pl.ds(tail0, tail)])
                pltpu.sync_copy(sm_s, ents_hbm.at[pl.ds(tail0, tail)])

            @pl.when(cid == 1)
            def _():
                pltpu.sync_copy(sm_c, relc_hbm.at[pl.ds(tail0, tail)])
                pltpu.sync_copy(sm_s, rels_hbm.at[pl.ds(tail0, tail)])

    return k(fc, bc, fe, be, src, dst, rel, zc, zs, ones_in)


def _tc_stats(u, v, r):
    n = u.shape[0]
    grid = n // BK

    def body(u_ref, v_ref, r_ref, o_ref):
        @pl.when(pl.program_id(0) == 0)
        def _():
            o_ref[...] = jnp.zeros_like(o_ref)
        for j, x_ref in enumerate((u_ref, v_ref, r_ref)):
            x = x_ref[...]
            o_ref[pl.ds(2 * j, 1), :] += jnp.sum(x, 0, keepdims=True)
            o_ref[pl.ds(2 * j + 1, 1), :] += jnp.sum(x * x, 0, keepdims=True)

    blk = pl.BlockSpec((BK, D), lambda i: (i, 0))
    return pl.pallas_call(
        body,
        grid=(grid,),
        in_specs=[blk, blk, blk],
        out_specs=pl.BlockSpec((8, D), lambda i: (0, 0)),
        out_shape=jax.ShapeDtypeStruct((8, D), jnp.float32),
    )(u, v, r)


def _tc_matmul(u, v, r, w0, w1, w2, bp):
    n = u.shape[0]
    grid = n // BK

    def body(u_ref, v_ref, r_ref, w0_ref, w1_ref, w2_ref, bp_ref,
             cf_ref, cb_ref, st_ref):
        uu = u_ref[...]
        vv = v_ref[...]
        rr = r_ref[...]
        w0m = w0_ref[...]
        w1m = w1_ref[...]
        s = jnp.dot(rr, w2_ref[...], preferred_element_type=jnp.float32)
        cf = (jnp.dot(uu, w0m, preferred_element_type=jnp.float32)
              + jnp.dot(vv, w1m, preferred_element_type=jnp.float32)
              + s + bp_ref[...])
        cb = (jnp.dot(vv, w0m, preferred_element_type=jnp.float32)
              + jnp.dot(uu, w1m, preferred_element_type=jnp.float32)
              - s + bp_ref[...])
        cf_ref[...] = cf
        cb_ref[...] = cb

        @pl.when(pl.program_id(0) == 0)
        def _():
            st_ref[...] = jnp.zeros_like(st_ref)
        st_ref[pl.ds(0, 1), :] += (jnp.sum(cf, 0, keepdims=True)
                                   + jnp.sum(cb, 0, keepdims=True))
        st_ref[pl.ds(1, 1), :] += (jnp.sum(cf * cf, 0, keepdims=True)
                                   + jnp.sum(cb * cb, 0, keepdims=True))

    blk = pl.BlockSpec((BK, D), lambda i: (i, 0))
    wblk = pl.BlockSpec((D, D), lambda i: (0, 0))
    return pl.pallas_call(
        body,
        grid=(grid,),
        in_specs=[blk, blk, blk, wblk, wblk, wblk,
                  pl.BlockSpec((1, D), lambda i: (0, 0))],
        out_specs=[blk, blk, pl.BlockSpec((8, D), lambda i: (0, 0))],
        out_shape=[jax.ShapeDtypeStruct((n, D), jnp.float32),
                   jax.ShapeDtypeStruct((n, D), jnp.float32),
                   jax.ShapeDtypeStruct((8, D), jnp.float32)],
    )(u, v, r, w0, w1, w2, bp)


def _tc_edge(cf, cb, par):
    n = cf.shape[0]
    grid = n // BK

    def half(c_ref, par_ref, oc_ref, oe_ref):
        c = c_ref[...]
        alpha = par_ref[pl.ds(0, 1), :]
        delta = par_ref[pl.ds(1, 1), :]
        w2v = par_ref[pl.ds(2, 1), :]
        b2 = par_ref[3, 0]
        a = jnp.sum(c * w2v, axis=1, keepdims=True) + b2
        e = jnp.exp(-jnp.where(a > 0, a, 0.01 * a))
        c2 = c * alpha + delta
        oc_ref[...] = e * c2
        oe_ref[...] = e

    def body(cf_ref, cb_ref, par_ref, fc_ref, bc_ref, fe_ref, be_ref):
        half(cf_ref, par_ref, fc_ref, fe_ref)
        half(cb_ref, par_ref, bc_ref, be_ref)

    blk = pl.BlockSpec((BK, D), lambda i: (i, 0))
    eblk = pl.BlockSpec((BK, 1), lambda i: (i, 0))
    return pl.pallas_call(
        body,
        grid=(grid,),
        in_specs=[blk, blk, pl.BlockSpec((8, D), lambda i: (0, 0))],
        out_specs=[blk, blk, eblk, eblk],
        out_shape=[jax.ShapeDtypeStruct((n, D), jnp.float32),
                   jax.ShapeDtypeStruct((n, D), jnp.float32),
                   jax.ShapeDtypeStruct((n, 1), jnp.float32),
                   jax.ShapeDtypeStruct((n, 1), jnp.float32)],
    )(cf, cb, par)


def _tc_divide(entc, ente, relc, rele):
    n_seg = entc.shape[0]
    grid = n_seg // BK

    def body(ec_ref, ee_ref, rc_ref, re_ref, he_ref, hr_ref):
        ebs = ee_ref[...]
        he_ref[...] = ec_ref[...] / jnp.where(ebs == 0.0, 1e-12, ebs)
        cnt = re_ref[...]
        hr_ref[...] = rc_ref[...] / jnp.maximum(cnt, 1.0)

    blk = pl.BlockSpec((BK, D), lambda i: (i, 0))
    eblk = pl.BlockSpec((BK, 1), lambda i: (i, 0))
    return pl.pallas_call(
        body,
        grid=(grid,),
        in_specs=[blk, eblk, blk, eblk],
        out_specs=[blk, blk],
        out_shape=[jax.ShapeDtypeStruct((n_seg, D), jnp.float32),
                   jax.ShapeDtypeStruct((n_seg, D), jnp.float32)],
    )(entc, ente, relc, rele)


@jax.jit
def kernel(triplets, ent_embed, rel_embed, W_a, b_a, W_a2, b_a2,
           gamma0, beta0, gamma1, beta1):
    n = triplets.shape[0]
    n_seg = ent_embed.shape[0]
    N = jnp.float32(2 * n)
    eps = jnp.float32(1e-5)

    src = triplets[:, 0]
    dst = triplets[:, 1]
    rel = triplets[:, 2]

    u, v, r = _sc_gather(ent_embed, rel_embed, src, dst, rel, n)

    st = _tc_stats(u, v, r)
    s01 = st[0] + st[2]
    q01 = st[1] + st[3]
    m01 = s01 / N
    var01 = q01 / N - m01 * m01
    var2 = 2.0 * st[5] / N
    m = jnp.concatenate([m01, m01, jnp.zeros_like(m01)])
    var = jnp.concatenate([var01, var01, var2])
    sfold = gamma0 * jax.lax.rsqrt(var + eps)
    Wp = W_a * sfold[None, :]
    bp = (b_a + W_a @ (beta0 - m * sfold)).reshape(1, D)
    w0 = Wp[:, 0:D].T
    w1 = Wp[:, D:2 * D].T
    w2 = Wp[:, 2 * D:3 * D].T

    cf, cb, cst = _tc_matmul(u, v, r, w0, w1, w2, bp)
    m2 = cst[0] / N
    v2 = cst[1] / N - m2 * m2
    alpha = gamma1 * jax.lax.rsqrt(v2 + eps)
    delta = beta1 - m2 * alpha
    w2v = W_a2[0] * alpha
    b2 = b_a2[0] + W_a2[0] @ delta
    par = jnp.zeros((8, D), jnp.float32)
    par = par.at[0].set(alpha).at[1].set(delta).at[2].set(w2v)
    par = par.at[3, 0].set(b2)

    fc, bc, fe, be = _tc_edge(cf, cb, par)
    fe1 = fe.reshape(n)
    be1 = be.reshape(n)

    zc = jnp.zeros((WB, D), jnp.float32)
    zs = jnp.zeros((GR,), jnp.float32)
    ones_in = jnp.ones((CH,), jnp.float32)
    entc, ents, relc, rels = _sc_scatter(fc, bc, fe1, be1,
                                         src, dst, rel, zc, zs, ones_in,
                                         n, n_seg)

    h_ent, h_rel = _tc_divide(entc, ents.reshape(n_seg, 1),
                              relc, rels.reshape(n_seg, 1))
    return (h_ent, h_rel)


# speedup vs baseline: 3.0270x; 1.8517x over previous
import functools
import jax
import jax.numpy as jnp
from jax import lax
from jax.experimental import pallas as pl
from jax.experimental.pallas import tpu as pltpu
from jax.experimental.pallas import tpu_sc as plsc

NC, NS = 2, 16
NW = NC * NS
CH = 128
D = 128
BK = 1000
GR = 624
WB = 48


def _sc_gather(ent_embed, rel_embed, src, dst, rel, n):
    nch = n // CH
    slots = (nch + NW - 1) // NW
    mesh = plsc.VectorSubcoreMesh(core_axis_name="c", subcore_axis_name="s")

    @functools.partial(
        pl.kernel, mesh=mesh,
        out_type=(jax.ShapeDtypeStruct((n, D), jnp.float32),) * 3,
        scratch_types=[
            pltpu.VMEM((CH,), jnp.int32),
            pltpu.VMEM((CH,), jnp.int32),
            pltpu.VMEM((CH,), jnp.int32),
            pltpu.VMEM((CH, D), jnp.float32),
            pltpu.VMEM((CH, D), jnp.float32),
            pltpu.VMEM((CH, D), jnp.float32),
        ] + [pltpu.SemaphoreType.DMA] * 9,
    )
    def k(ent_hbm, rel_hbm, src_hbm, dst_hbm, relid_hbm,
          u_hbm, v_hbm, r_hbm, iu, iv, ir, bu, bv, br,
          s1, s2, s3, s4, s5, s6, s7, s8, s9):
        wid = lax.axis_index("s") * NC + lax.axis_index("c")

        def body(i, carry):
            c = i * NW + wid

            @pl.when(c < nch)
            def _():
                row0 = c * CH
                hiu = pltpu.async_copy(src_hbm.at[pl.ds(row0, CH)], iu, s1)
                hiv = pltpu.async_copy(dst_hbm.at[pl.ds(row0, CH)], iv, s2)
                hir = pltpu.async_copy(relid_hbm.at[pl.ds(row0, CH)], ir, s3)
                hiu.wait()
                hgu = pltpu.async_copy(ent_hbm.at[iu], bu, s4)
                hiv.wait()
                hgv = pltpu.async_copy(ent_hbm.at[iv], bv, s5)
                hir.wait()
                hgr = pltpu.async_copy(rel_hbm.at[ir], br, s6)
                hgu.wait()
                hwu = pltpu.async_copy(bu, u_hbm.at[pl.ds(row0, CH)], s7)
                hgv.wait()
                hwv = pltpu.async_copy(bv, v_hbm.at[pl.ds(row0, CH)], s8)
                hgr.wait()
                hwr = pltpu.async_copy(br, r_hbm.at[pl.ds(row0, CH)], s9)
                hwu.wait()
                hwv.wait()
                hwr.wait()

            return carry

        lax.fori_loop(0, slots, body, 0)

    return k(ent_embed, rel_embed, src, dst, rel)


def _sc_scatter(fc, bc, fe, be, src, dst, rel, zc, zs, ones_in, n, n_seg):
    nch = n // CH
    slots = (nch + NS - 1) // NS
    tail0 = NS * GR
    tail = n_seg - tail0
    mesh = plsc.VectorSubcoreMesh(core_axis_name="c", subcore_axis_name="s")

    @functools.partial(
        pl.kernel, mesh=mesh,
        out_type=(jax.ShapeDtypeStruct((n_seg, D), jnp.float32),
                  jax.ShapeDtypeStruct((n_seg,), jnp.float32),
                  jax.ShapeDtypeStruct((n_seg, D), jnp.float32),
                  jax.ShapeDtypeStruct((n_seg,), jnp.float32)),
        scratch_types=[
            pltpu.VMEM_SHARED((n_seg, D), jnp.float32),
            pltpu.VMEM_SHARED((n_seg,), jnp.float32),
            pltpu.VMEM((CH,), jnp.int32),
            pltpu.VMEM((CH, D), jnp.float32),
            pltpu.VMEM((CH,), jnp.float32),
            pltpu.VMEM((CH,), jnp.float32),
            pltpu.VMEM((WB, D), jnp.float32),
            pltpu.VMEM((GR,), jnp.float32),
            pltpu.VMEM((16, D), jnp.float32),
            pltpu.VMEM((16,), jnp.float32),
        ] + [pltpu.SemaphoreType.DMA] * 3,
    )
    def k(fc_hbm, bc_hbm, fe_hbm, be_hbm, src_hbm, dst_hbm, rel_hbm,
          zc_hbm, zs_hbm, ones_hbm,
          entc_hbm, ents_hbm, relc_hbm, rels_hbm,
          acc_c, acc_s, idx_v, rows_v, e_v, ones_v, big_c, srow_v,
          sm_c, sm_s, s1, s2, s3):
        cid = lax.axis_index("c")
        sid = lax.axis_index("s")
        r0 = sid * GR

        pltpu.sync_copy(zc_hbm, big_c)
        for j in range(GR // WB):
            pltpu.sync_copy(big_c, acc_c.at[pl.ds(r0 + j * WB, WB)])
        pltpu.sync_copy(zs_hbm, srow_v)
        pltpu.sync_copy(srow_v, acc_s.at[pl.ds(r0, GR)])
        pltpu.sync_copy(ones_hbm, ones_v)

        @pl.when(sid == 0)
        def _():
            pltpu.sync_copy(zc_hbm.at[pl.ds(0, tail)], sm_c)
            pltpu.sync_copy(sm_c, acc_c.at[pl.ds(tail0, tail)])
            pltpu.sync_copy(zs_hbm.at[pl.ds(0, tail)], sm_s)
            pltpu.sync_copy(sm_s, acc_s.at[pl.ds(tail0, tail)])

        plsc.subcore_barrier()

        def scan_ent(data_c, data_e, ids):
            def body(i, carry):
                c = i * NS + sid

                @pl.when(c < nch)
                def _():
                    row0 = c * CH
                    h1 = pltpu.async_copy(ids.at[pl.ds(row0, CH)], idx_v, s1)
                    h2 = pltpu.async_copy(data_c.at[pl.ds(row0, CH)], rows_v, s2)
                    h3 = pltpu.async_copy(data_e.at[pl.ds(row0, CH)], e_v, s3)
                    h1.wait()
                    h2.wait()
                    h3.wait()
                    pltpu.sync_copy(rows_v, acc_c.at[idx_v], add=True)
                    pltpu.sync_copy(e_v, acc_s.at[idx_v], add=True)

                return carry
            lax.fori_loop(0, slots, body, 0)

        def scan_rel(data_c, ids):
            def body(i, carry):
                c = i * NS + sid

                @pl.when(c < nch)
                def _():
                    row0 = c * CH
                    h1 = pltpu.async_copy(ids.at[pl.ds(row0, CH)], idx_v, s1)
                    h2 = pltpu.async_copy(data_c.at[pl.ds(row0, CH)], rows_v, s2)
                    h1.wait()
                    h2.wait()
                    pltpu.sync_copy(rows_v, acc_c.at[idx_v], add=True)
                    pltpu.sync_copy(ones_v, acc_s.at[idx_v], add=True)

                return carry
            lax.fori_loop(0, slots, body, 0)

        @pl.when(cid == 0)
        def _():
            scan_ent(fc_hbm, fe_hbm, src_hbm)
            scan_ent(bc_hbm, be_hbm, dst_hbm)

        @pl.when(cid == 1)
        def _():
            scan_rel(fc_hbm, rel_hbm)

        plsc.subcore_barrier()

        for j in range(GR // WB):
            rj = r0 + j * WB
            pltpu.sync_copy(acc_c.at[pl.ds(rj, WB)], big_c)

            @pl.when(cid == 0)
            def _():
                pltpu.sync_copy(big_c, entc_hbm.at[pl.ds(rj, WB)])

            @pl.when(cid == 1)
            def _():
                pltpu.sync_copy(big_c, relc_hbm.at[pl.ds(rj, WB)])

        pltpu.sync_copy(acc_s.at[pl.ds(r0, GR)], srow_v)

        @pl.when(cid == 0)
        def _():
            pltpu.sync_copy(srow_v, ents_hbm.at[pl.ds(r0, GR)])

        @pl.when(cid == 1)
        def _():
            pltpu.sync_copy(srow_v, rels_hbm.at[pl.ds(r0, GR)])

        @pl.when(sid == 0)
        def _():
            pltpu.sync_copy(acc_c.at[pl.ds(tail0, tail)], sm_c)
            pltpu.sync_copy(acc_s.at[pl.ds(tail0, tail)], sm_s)

            @pl.when(cid == 0)
            def _():
                pltpu.sync_copy(sm_c, entc_hbm.at[pl.ds(tail0, tail)])
                pltpu.sync_copy(sm_s, ents_hbm.at[pl.ds(tail0, tail)])

            @pl.when(cid == 1)
            def _():
                pltpu.sync_copy(sm_c, relc_hbm.at[pl.ds(tail0, tail)])
                pltpu.sync_copy(sm_s, rels_hbm.at[pl.ds(tail0, tail)])

    return k(fc, bc, fe, be, src, dst, rel, zc, zs, ones_in)


def _tc_stats(u, v, r):
    n = u.shape[0]
    grid = n // BK

    def body(u_ref, v_ref, r_ref, o_ref):
        @pl.when(pl.program_id(0) == 0)
        def _():
            o_ref[...] = jnp.zeros_like(o_ref)
        for j, x_ref in enumerate((u_ref, v_ref, r_ref)):
            x = x_ref[...]
            o_ref[pl.ds(2 * j, 1), :] += jnp.sum(x, 0, keepdims=True)
            o_ref[pl.ds(2 * j + 1, 1), :] += jnp.sum(x * x, 0, keepdims=True)

    blk = pl.BlockSpec((BK, D), lambda i: (i, 0))
    return pl.pallas_call(
        body,
        grid=(grid,),
        in_specs=[blk, blk, blk],
        out_specs=pl.BlockSpec((8, D), lambda i: (0, 0)),
        out_shape=jax.ShapeDtypeStruct((8, D), jnp.float32),
    )(u, v, r)


def _tc_matmul(u, v, r, w0, w1, w2, bp):
    n = u.shape[0]
    grid = n // BK

    def body(u_ref, v_ref, r_ref, w0_ref, w1_ref, w2_ref, bp_ref,
             cf_ref, cb_ref, st_ref):
        uu = u_ref[...]
        vv = v_ref[...]
        rr = r_ref[...]
        w0m = w0_ref[...]
        w1m = w1_ref[...]
        s = jnp.dot(rr, w2_ref[...], preferred_element_type=jnp.float32)
        cf = (jnp.dot(uu, w0m, preferred_element_type=jnp.float32)
              + jnp.dot(vv, w1m, preferred_element_type=jnp.float32)
              + s + bp_ref[...])
        cb = (jnp.dot(vv, w0m, preferred_element_type=jnp.float32)
              + jnp.dot(uu, w1m, preferred_element_type=jnp.float32)
              - s + bp_ref[...])
        cf_ref[...] = cf
        cb_ref[...] = cb

        @pl.when(pl.program_id(0) == 0)
        def _():
            st_ref[...] = jnp.zeros_like(st_ref)
        st_ref[pl.ds(0, 1), :] += (jnp.sum(cf, 0, keepdims=True)
                                   + jnp.sum(cb, 0, keepdims=True))
        st_ref[pl.ds(1, 1), :] += (jnp.sum(cf * cf, 0, keepdims=True)
                                   + jnp.sum(cb * cb, 0, keepdims=True))

    blk = pl.BlockSpec((BK, D), lambda i: (i, 0))
    wblk = pl.BlockSpec((D, D), lambda i: (0, 0))
    return pl.pallas_call(
        body,
        grid=(grid,),
        in_specs=[blk, blk, blk, wblk, wblk, wblk,
                  pl.BlockSpec((1, D), lambda i: (0, 0))],
        out_specs=[blk, blk, pl.BlockSpec((8, D), lambda i: (0, 0))],
        out_shape=[jax.ShapeDtypeStruct((n, D), jnp.float32),
                   jax.ShapeDtypeStruct((n, D), jnp.float32),
                   jax.ShapeDtypeStruct((8, D), jnp.float32)],
    )(u, v, r, w0, w1, w2, bp)


def _tc_edge(cf, cb, par):
    n = cf.shape[0]
    grid = n // BK

    def half(c_ref, par_ref, oc_ref, oe_ref):
        c = c_ref[...]
        alpha = par_ref[pl.ds(0, 1), :]
        delta = par_ref[pl.ds(1, 1), :]
        w2v = par_ref[pl.ds(2, 1), :]
        b2 = par_ref[3, 0]
        a = jnp.sum(c * w2v, axis=1, keepdims=True) + b2
        e = jnp.exp(-jnp.where(a > 0, a, 0.01 * a))
        c2 = c * alpha + delta
        oc_ref[...] = e * c2
        oe_ref[...] = e

    def body(cf_ref, cb_ref, par_ref, fc_ref, bc_ref, fe_ref, be_ref):
        half(cf_ref, par_ref, fc_ref, fe_ref)
        half(cb_ref, par_ref, bc_ref, be_ref)

    blk = pl.BlockSpec((BK, D), lambda i: (i, 0))
    eblk = pl.BlockSpec((BK, 1), lambda i: (i, 0))
    return pl.pallas_call(
        body,
        grid=(grid,),
        in_specs=[blk, blk, pl.BlockSpec((8, D), lambda i: (0, 0))],
        out_specs=[blk, blk, eblk, eblk],
        out_shape=[jax.ShapeDtypeStruct((n, D), jnp.float32),
                   jax.ShapeDtypeStruct((n, D), jnp.float32),
                   jax.ShapeDtypeStruct((n, 1), jnp.float32),
                   jax.ShapeDtypeStruct((n, 1), jnp.float32)],
    )(cf, cb, par)


def _tc_divide(entc, ente, relc, rele):
    n_seg = entc.shape[0]
    grid = n_seg // BK

    def body(ec_ref, ee_ref, rc_ref, re_ref, he_ref, hr_ref):
        ebs = ee_ref[...]
        he_ref[...] = ec_ref[...] / jnp.where(ebs == 0.0, 1e-12, ebs)
        cnt = re_ref[...]
        hr_ref[...] = rc_ref[...] / jnp.maximum(cnt, 1.0)

    blk = pl.BlockSpec((BK, D), lambda i: (i, 0))
    eblk = pl.BlockSpec((BK, 1), lambda i: (i, 0))
    return pl.pallas_call(
        body,
        grid=(grid,),
        in_specs=[blk, eblk, blk, eblk],
        out_specs=[blk, blk],
        out_shape=[jax.ShapeDtypeStruct((n_seg, D), jnp.float32),
                   jax.ShapeDtypeStruct((n_seg, D), jnp.float32)],
    )(entc, ente, relc, rele)


@jax.jit
def kernel(triplets, ent_embed, rel_embed, W_a, b_a, W_a2, b_a2,
           gamma0, beta0, gamma1, beta1):
    n = triplets.shape[0]
    n_seg = ent_embed.shape[0]
    N = jnp.float32(2 * n)
    eps = jnp.float32(1e-5)

    src = triplets[:, 0]
    dst = triplets[:, 1]
    rel = triplets[:, 2]

    u, v, r = _sc_gather(ent_embed, rel_embed, src, dst, rel, n)

    st = _tc_stats(u, v, r)
    s01 = st[0] + st[2]
    q01 = st[1] + st[3]
    m01 = s01 / N
    var01 = q01 / N - m01 * m01
    var2 = 2.0 * st[5] / N
    m = jnp.concatenate([m01, m01, jnp.zeros_like(m01)])
    var = jnp.concatenate([var01, var01, var2])
    sfold = gamma0 * jax.lax.rsqrt(var + eps)
    Wp = W_a * sfold[None, :]
    bp = (b_a + W_a @ (beta0 - m * sfold)).reshape(1, D)
    w0 = Wp[:, 0:D].T
    w1 = Wp[:, D:2 * D].T
    w2 = Wp[:, 2 * D:3 * D].T

    cf, cb, cst = _tc_matmul(u, v, r, w0, w1, w2, bp)
    m2 = cst[0] / N
    v2 = cst[1] / N - m2 * m2
    alpha = gamma1 * jax.lax.rsqrt(v2 + eps)
    delta = beta1 - m2 * alpha
    w2v = W_a2[0] * alpha
    b2 = b_a2[0] + W_a2[0] @ delta
    par = jnp.zeros((8, D), jnp.float32)
    par = par.at[0].set(alpha).at[1].set(delta).at[2].set(w2v)
    par = par.at[3, 0].set(b2)

    fc, bc, fe, be = _tc_edge(cf, cb, par)
    fe1 = fe.reshape(n)
    be1 = be.reshape(n)

    zc = jnp.zeros((WB, D), jnp.float32)
    zs = jnp.zeros((GR,), jnp.float32)
    ones_in = jnp.ones((CH,), jnp.float32)
    entc, ents, relc, rels = _sc_scatter(fc, bc, fe1, be1,
                                         src, dst, rel, zc, zs, ones_in,
                                         n, n_seg)

    h_ent, h_rel = _tc_divide(entc, ents.reshape(n_seg, 1),
                              relc, rels.reshape(n_seg, 1))
    return (h_ent, h_rel)


# speedup vs baseline: 3.0495x; 1.0074x over previous
import functools
import jax
import jax.numpy as jnp
from jax import lax
from jax.experimental import pallas as pl
from jax.experimental.pallas import tpu as pltpu
from jax.experimental.pallas import tpu_sc as plsc

NC, NS = 2, 16
NW = NC * NS
CH = 128
D = 128
BK = 1000
GR = 624
WB = 48


def _sc_gather(ent_embed, rel_embed, src, dst, rel, n):
    nch = n // CH
    slots = (nch + NW - 1) // NW
    mesh = plsc.VectorSubcoreMesh(core_axis_name="c", subcore_axis_name="s")

    @functools.partial(
        pl.kernel, mesh=mesh,
        out_type=(jax.ShapeDtypeStruct((n, D), jnp.float32),) * 3,
        scratch_types=[
            pltpu.VMEM((CH,), jnp.int32),
            pltpu.VMEM((CH,), jnp.int32),
            pltpu.VMEM((CH,), jnp.int32),
            pltpu.VMEM((CH, D), jnp.float32),
            pltpu.VMEM((CH, D), jnp.float32),
            pltpu.VMEM((CH, D), jnp.float32),
        ] + [pltpu.SemaphoreType.DMA] * 9,
    )
    def k(ent_hbm, rel_hbm, src_hbm, dst_hbm, relid_hbm,
          u_hbm, v_hbm, r_hbm, iu, iv, ir, bu, bv, br,
          s1, s2, s3, s4, s5, s6, s7, s8, s9):
        wid = lax.axis_index("s") * NC + lax.axis_index("c")

        def body(i, carry):
            c = i * NW + wid

            @pl.when(c < nch)
            def _():
                row0 = c * CH
                hiu = pltpu.async_copy(src_hbm.at[pl.ds(row0, CH)], iu, s1)
                hiv = pltpu.async_copy(dst_hbm.at[pl.ds(row0, CH)], iv, s2)
                hir = pltpu.async_copy(relid_hbm.at[pl.ds(row0, CH)], ir, s3)
                hiu.wait()
                hgu = pltpu.async_copy(ent_hbm.at[iu], bu, s4)
                hiv.wait()
                hgv = pltpu.async_copy(ent_hbm.at[iv], bv, s5)
                hir.wait()
                hgr = pltpu.async_copy(rel_hbm.at[ir], br, s6)
                hgu.wait()
                hwu = pltpu.async_copy(bu, u_hbm.at[pl.ds(row0, CH)], s7)
                hgv.wait()
                hwv = pltpu.async_copy(bv, v_hbm.at[pl.ds(row0, CH)], s8)
                hgr.wait()
                hwr = pltpu.async_copy(br, r_hbm.at[pl.ds(row0, CH)], s9)
                hwu.wait()
                hwv.wait()
                hwr.wait()

            return carry

        lax.fori_loop(0, slots, body, 0)

    return k(ent_embed, rel_embed, src, dst, rel)


def _sc_scatter(fc, bc, fe, be, src, dst, rel, zc, zs, ones_in, n, n_seg):
    nch = n // CH
    slots = (nch + NS - 1) // NS
    tail0 = NS * GR
    tail = n_seg - tail0
    mesh = plsc.VectorSubcoreMesh(core_axis_name="c", subcore_axis_name="s")

    @functools.partial(
        pl.kernel, mesh=mesh,
        out_type=(jax.ShapeDtypeStruct((n_seg, D), jnp.float32),
                  jax.ShapeDtypeStruct((n_seg,), jnp.float32),
                  jax.ShapeDtypeStruct((n_seg, D), jnp.float32),
                  jax.ShapeDtypeStruct((n_seg,), jnp.float32)),
        scratch_types=[
            pltpu.VMEM_SHARED((n_seg, D), jnp.float32),
            pltpu.VMEM_SHARED((n_seg,), jnp.float32),
            pltpu.VMEM((CH,), jnp.int32),
            pltpu.VMEM((CH, D), jnp.float32),
            pltpu.VMEM((CH,), jnp.float32),
            pltpu.VMEM((CH,), jnp.float32),
            pltpu.VMEM((WB, D), jnp.float32),
            pltpu.VMEM((GR,), jnp.float32),
            pltpu.VMEM((16, D), jnp.float32),
            pltpu.VMEM((16,), jnp.float32),
        ] + [pltpu.SemaphoreType.DMA] * 3,
    )
    def k(fc_hbm, bc_hbm, fe_hbm, be_hbm, src_hbm, dst_hbm, rel_hbm,
          zc_hbm, zs_hbm, ones_hbm,
          entc_hbm, ents_hbm, relc_hbm, rels_hbm,
          acc_c, acc_s, idx_v, rows_v, e_v, ones_v, big_c, srow_v,
          sm_c, sm_s, s1, s2, s3):
        cid = lax.axis_index("c")
        sid = lax.axis_index("s")
        r0 = sid * GR

        pltpu.sync_copy(zc_hbm, big_c)
        for j in range(GR // WB):
            pltpu.sync_copy(big_c, acc_c.at[pl.ds(r0 + j * WB, WB)])
        pltpu.sync_copy(zs_hbm, srow_v)
        pltpu.sync_copy(srow_v, acc_s.at[pl.ds(r0, GR)])
        pltpu.sync_copy(ones_hbm, ones_v)

        @pl.when(sid == 0)
        def _():
            pltpu.sync_copy(zc_hbm.at[pl.ds(0, tail)], sm_c)
            pltpu.sync_copy(sm_c, acc_c.at[pl.ds(tail0, tail)])
            pltpu.sync_copy(zs_hbm.at[pl.ds(0, tail)], sm_s)
            pltpu.sync_copy(sm_s, acc_s.at[pl.ds(tail0, tail)])

        plsc.subcore_barrier()

        def scan_ent(data_c, data_e, ids):
            def body(i, carry):
                c = i * NS + sid

                @pl.when(c < nch)
                def _():
                    row0 = c * CH
                    h1 = pltpu.async_copy(ids.at[pl.ds(row0, CH)], idx_v, s1)
                    h2 = pltpu.async_copy(data_c.at[pl.ds(row0, CH)], rows_v, s2)
                    h3 = pltpu.async_copy(data_e.at[pl.ds(row0, CH)], e_v, s3)
                    h1.wait()
                    h2.wait()
                    h3.wait()
                    g1 = pltpu.async_copy(rows_v, acc_c.at[idx_v], s2,
                                          add=True)
                    g2 = pltpu.async_copy(e_v, acc_s.at[idx_v], s3, add=True)
                    g1.wait()
                    g2.wait()

                return carry
            lax.fori_loop(0, slots, body, 0)

        def scan_rel(data_c, ids):
            def body(i, carry):
                c = i * NS + sid

                @pl.when(c < nch)
                def _():
                    row0 = c * CH
                    h1 = pltpu.async_copy(ids.at[pl.ds(row0, CH)], idx_v, s1)
                    h2 = pltpu.async_copy(data_c.at[pl.ds(row0, CH)], rows_v, s2)
                    h1.wait()
                    h2.wait()
                    g1 = pltpu.async_copy(rows_v, acc_c.at[idx_v], s2,
                                          add=True)
                    g2 = pltpu.async_copy(ones_v, acc_s.at[idx_v], s3,
                                          add=True)
                    g1.wait()
                    g2.wait()

                return carry
            lax.fori_loop(0, slots, body, 0)

        @pl.when(cid == 0)
        def _():
            scan_ent(fc_hbm, fe_hbm, src_hbm)
            scan_ent(bc_hbm, be_hbm, dst_hbm)

        @pl.when(cid == 1)
        def _():
            scan_rel(fc_hbm, rel_hbm)

        plsc.subcore_barrier()

        for j in range(GR // WB):
            rj = r0 + j * WB
            pltpu.sync_copy(acc_c.at[pl.ds(rj, WB)], big_c)

            @pl.when(cid == 0)
            def _():
                pltpu.sync_copy(big_c, entc_hbm.at[pl.ds(rj, WB)])

            @pl.when(cid == 1)
            def _():
                pltpu.sync_copy(big_c, relc_hbm.at[pl.ds(rj, WB)])

        pltpu.sync_copy(acc_s.at[pl.ds(r0, GR)], srow_v)

        @pl.when(cid == 0)
        def _():
            pltpu.sync_copy(srow_v, ents_hbm.at[pl.ds(r0, GR)])

        @pl.when(cid == 1)
        def _():
            pltpu.sync_copy(srow_v, rels_hbm.at[pl.ds(r0, GR)])

        @pl.when(sid == 0)
        def _():
            pltpu.sync_copy(acc_c.at[pl.ds(tail0, tail)], sm_c)
            pltpu.sync_copy(acc_s.at[pl.ds(tail0, tail)], sm_s)

            @pl.when(cid == 0)
            def _():
                pltpu.sync_copy(sm_c, entc_hbm.at[pl.ds(tail0, tail)])
                pltpu.sync_copy(sm_s, ents_hbm.at[pl.ds(tail0, tail)])

            @pl.when(cid == 1)
            def _():
                pltpu.sync_copy(sm_c, relc_hbm.at[pl.ds(tail0, tail)])
                pltpu.sync_copy(sm_s, rels_hbm.at[pl.ds(tail0, tail)])

    return k(fc, bc, fe, be, src, dst, rel, zc, zs, ones_in)


def _tc_stats(u, v, r):
    n = u.shape[0]
    grid = n // BK

    def body(u_ref, v_ref, r_ref, o_ref):
        @pl.when(pl.program_id(0) == 0)
        def _():
            o_ref[...] = jnp.zeros_like(o_ref)
        for j, x_ref in enumerate((u_ref, v_ref, r_ref)):
            x = x_ref[...]
            o_ref[pl.ds(2 * j, 1), :] += jnp.sum(x, 0, keepdims=True)
            o_ref[pl.ds(2 * j + 1, 1), :] += jnp.sum(x * x, 0, keepdims=True)

    blk = pl.BlockSpec((BK, D), lambda i: (i, 0))
    return pl.pallas_call(
        body,
        grid=(grid,),
        in_specs=[blk, blk, blk],
        out_specs=pl.BlockSpec((8, D), lambda i: (0, 0)),
        out_shape=jax.ShapeDtypeStruct((8, D), jnp.float32),
    )(u, v, r)


def _tc_matmul(u, v, r, w0, w1, w2, bp):
    n = u.shape[0]
    grid = n // BK

    def body(u_ref, v_ref, r_ref, w0_ref, w1_ref, w2_ref, bp_ref,
             cf_ref, cb_ref, st_ref):
        uu = u_ref[...]
        vv = v_ref[...]
        rr = r_ref[...]
        w0m = w0_ref[...]
        w1m = w1_ref[...]
        s = jnp.dot(rr, w2_ref[...], preferred_element_type=jnp.float32)
        cf = (jnp.dot(uu, w0m, preferred_element_type=jnp.float32)
              + jnp.dot(vv, w1m, preferred_element_type=jnp.float32)
              + s + bp_ref[...])
        cb = (jnp.dot(vv, w0m, preferred_element_type=jnp.float32)
              + jnp.dot(uu, w1m, preferred_element_type=jnp.float32)
              - s + bp_ref[...])
        cf_ref[...] = cf
        cb_ref[...] = cb

        @pl.when(pl.program_id(0) == 0)
        def _():
            st_ref[...] = jnp.zeros_like(st_ref)
        st_ref[pl.ds(0, 1), :] += (jnp.sum(cf, 0, keepdims=True)
                                   + jnp.sum(cb, 0, keepdims=True))
        st_ref[pl.ds(1, 1), :] += (jnp.sum(cf * cf, 0, keepdims=True)
                                   + jnp.sum(cb * cb, 0, keepdims=True))

    blk = pl.BlockSpec((BK, D), lambda i: (i, 0))
    wblk = pl.BlockSpec((D, D), lambda i: (0, 0))
    return pl.pallas_call(
        body,
        grid=(grid,),
        in_specs=[blk, blk, blk, wblk, wblk, wblk,
                  pl.BlockSpec((1, D), lambda i: (0, 0))],
        out_specs=[blk, blk, pl.BlockSpec((8, D), lambda i: (0, 0))],
        out_shape=[jax.ShapeDtypeStruct((n, D), jnp.float32),
                   jax.ShapeDtypeStruct((n, D), jnp.float32),
                   jax.ShapeDtypeStruct((8, D), jnp.float32)],
    )(u, v, r, w0, w1, w2, bp)


def _tc_edge(cf, cb, par):
    n = cf.shape[0]
    grid = n // BK

    def half(c_ref, par_ref, oc_ref, oe_ref):
        c = c_ref[...]
        alpha = par_ref[pl.ds(0, 1), :]
        delta = par_ref[pl.ds(1, 1), :]
        w2v = par_ref[pl.ds(2, 1), :]
        b2 = par_ref[3, 0]
        a = jnp.sum(c * w2v, axis=1, keepdims=True) + b2
        e = jnp.exp(-jnp.where(a > 0, a, 0.01 * a))
        c2 = c * alpha + delta
        oc_ref[...] = e * c2
        oe_ref[...] = e

    def body(cf_ref, cb_ref, par_ref, fc_ref, bc_ref, fe_ref, be_ref):
        half(cf_ref, par_ref, fc_ref, fe_ref)
        half(cb_ref, par_ref, bc_ref, be_ref)

    blk = pl.BlockSpec((BK, D), lambda i: (i, 0))
    eblk = pl.BlockSpec((BK, 1), lambda i: (i, 0))
    return pl.pallas_call(
        body,
        grid=(grid,),
        in_specs=[blk, blk, pl.BlockSpec((8, D), lambda i: (0, 0))],
        out_specs=[blk, blk, eblk, eblk],
        out_shape=[jax.ShapeDtypeStruct((n, D), jnp.float32),
                   jax.ShapeDtypeStruct((n, D), jnp.float32),
                   jax.ShapeDtypeStruct((n, 1), jnp.float32),
                   jax.ShapeDtypeStruct((n, 1), jnp.float32)],
    )(cf, cb, par)


def _tc_divide(entc, ente, relc, rele):
    n_seg = entc.shape[0]
    grid = n_seg // BK

    def body(ec_ref, ee_ref, rc_ref, re_ref, he_ref, hr_ref):
        ebs = ee_ref[...]
        he_ref[...] = ec_ref[...] / jnp.where(ebs == 0.0, 1e-12, ebs)
        cnt = re_ref[...]
        hr_ref[...] = rc_ref[...] / jnp.maximum(cnt, 1.0)

    blk = pl.BlockSpec((BK, D), lambda i: (i, 0))
    eblk = pl.BlockSpec((BK, 1), lambda i: (i, 0))
    return pl.pallas_call(
        body,
        grid=(grid,),
        in_specs=[blk, eblk, blk, eblk],
        out_specs=[blk, blk],
        out_shape=[jax.ShapeDtypeStruct((n_seg, D), jnp.float32),
                   jax.ShapeDtypeStruct((n_seg, D), jnp.float32)],
    )(entc, ente, relc, rele)


@jax.jit
def kernel(triplets, ent_embed, rel_embed, W_a, b_a, W_a2, b_a2,
           gamma0, beta0, gamma1, beta1):
    n = triplets.shape[0]
    n_seg = ent_embed.shape[0]
    N = jnp.float32(2 * n)
    eps = jnp.float32(1e-5)

    src = triplets[:, 0]
    dst = triplets[:, 1]
    rel = triplets[:, 2]

    u, v, r = _sc_gather(ent_embed, rel_embed, src, dst, rel, n)

    st = _tc_stats(u, v, r)
    s01 = st[0] + st[2]
    q01 = st[1] + st[3]
    m01 = s01 / N
    var01 = q01 / N - m01 * m01
    var2 = 2.0 * st[5] / N
    m = jnp.concatenate([m01, m01, jnp.zeros_like(m01)])
    var = jnp.concatenate([var01, var01, var2])
    sfold = gamma0 * jax.lax.rsqrt(var + eps)
    Wp = W_a * sfold[None, :]
    bp = (b_a + W_a @ (beta0 - m * sfold)).reshape(1, D)
    w0 = Wp[:, 0:D].T
    w1 = Wp[:, D:2 * D].T
    w2 = Wp[:, 2 * D:3 * D].T

    cf, cb, cst = _tc_matmul(u, v, r, w0, w1, w2, bp)
    m2 = cst[0] / N
    v2 = cst[1] / N - m2 * m2
    alpha = gamma1 * jax.lax.rsqrt(v2 + eps)
    delta = beta1 - m2 * alpha
    w2v = W_a2[0] * alpha
    b2 = b_a2[0] + W_a2[0] @ delta
    par = jnp.zeros((8, D), jnp.float32)
    par = par.at[0].set(alpha).at[1].set(delta).at[2].set(w2v)
    par = par.at[3, 0].set(b2)

    fc, bc, fe, be = _tc_edge(cf, cb, par)
    fe1 = fe.reshape(n)
    be1 = be.reshape(n)

    zc = jnp.zeros((WB, D), jnp.float32)
    zs = jnp.zeros((GR,), jnp.float32)
    ones_in = jnp.ones((CH,), jnp.float32)
    entc, ents, relc, rels = _sc_scatter(fc, bc, fe1, be1,
                                         src, dst, rel, zc, zs, ones_in,
                                         n, n_seg)

    h_ent, h_rel = _tc_divide(entc, ents.reshape(n_seg, 1),
                              relc, rels.reshape(n_seg, 1))
    return (h_ent, h_rel)


# speedup vs baseline: 3.2503x; 1.0658x over previous
import functools
import jax
import jax.numpy as jnp
from jax import lax
from jax.experimental import pallas as pl
from jax.experimental.pallas import tpu as pltpu
from jax.experimental.pallas import tpu_sc as plsc

NC, NS = 2, 16
NW = NC * NS
CH = 128
D = 128
BK = 1000
GR = 624
WB = 48
TAIL0 = NS * GR
TAIL = 16


def _sc_gather(ent_embed, rel_embed, src, dst, rel, zs, ones_in, n, n_seg):
    nch = n // CH
    slots = (nch + NW - 1) // NW
    mesh = plsc.VectorSubcoreMesh(core_axis_name="c", subcore_axis_name="s")

    @functools.partial(
        pl.kernel, mesh=mesh,
        out_type=(jax.ShapeDtypeStruct((n, D), jnp.float32),) * 3
        + (jax.ShapeDtypeStruct((n_seg,), jnp.float32),) * 6,
        scratch_types=[
            pltpu.VMEM_SHARED((n_seg,), jnp.float32),
            pltpu.VMEM_SHARED((n_seg,), jnp.float32),
            pltpu.VMEM_SHARED((n_seg,), jnp.float32),
            pltpu.VMEM((CH,), jnp.int32),
            pltpu.VMEM((CH,), jnp.int32),
            pltpu.VMEM((CH,), jnp.int32),
            pltpu.VMEM((CH, D), jnp.float32),
            pltpu.VMEM((CH, D), jnp.float32),
            pltpu.VMEM((CH, D), jnp.float32),
            pltpu.VMEM((CH,), jnp.float32),
            pltpu.VMEM((GR,), jnp.float32),
            pltpu.VMEM((16,), jnp.float32),
        ] + [pltpu.SemaphoreType.DMA] * 12,
    )
    def k(ent_hbm, rel_hbm, src_hbm, dst_hbm, relid_hbm, zs_hbm, ones_hbm,
          u_hbm, v_hbm, r_hbm, cs0_hbm, cs1_hbm, cd0_hbm, cd1_hbm,
          cr0_hbm, cr1_hbm,
          a_src, a_dst, a_rel, iu, iv, ir, bu, bv, br, ones_v, srow_v, sm_s,
          s1, s2, s3, s4, s5, s6, s7, s8, s9, sa, sb, sc):
        cid = lax.axis_index("c")
        sid = lax.axis_index("s")
        wid = sid * NC + cid
        r0 = sid * GR

        pltpu.sync_copy(zs_hbm, srow_v)
        pltpu.sync_copy(srow_v, a_src.at[pl.ds(r0, GR)])
        pltpu.sync_copy(srow_v, a_dst.at[pl.ds(r0, GR)])
        pltpu.sync_copy(srow_v, a_rel.at[pl.ds(r0, GR)])
        pltpu.sync_copy(ones_hbm, ones_v)

        @pl.when(sid == 0)
        def _():
            pltpu.sync_copy(zs_hbm.at[pl.ds(0, TAIL)], sm_s)
            pltpu.sync_copy(sm_s, a_src.at[pl.ds(TAIL0, TAIL)])
            pltpu.sync_copy(sm_s, a_dst.at[pl.ds(TAIL0, TAIL)])
            pltpu.sync_copy(sm_s, a_rel.at[pl.ds(TAIL0, TAIL)])

        plsc.subcore_barrier()

        def body(i, carry):
            c = i * NW + wid

            @pl.when(c < nch)
            def _():
                row0 = c * CH
                hiu = pltpu.async_copy(src_hbm.at[pl.ds(row0, CH)], iu, s1)
                hiv = pltpu.async_copy(dst_hbm.at[pl.ds(row0, CH)], iv, s2)
                hir = pltpu.async_copy(relid_hbm.at[pl.ds(row0, CH)], ir, s3)
                hiu.wait()
                hgu = pltpu.async_copy(ent_hbm.at[iu], bu, s4)
                hc1 = pltpu.async_copy(ones_v, a_src.at[iu], sa, add=True)
                hiv.wait()
                hgv = pltpu.async_copy(ent_hbm.at[iv], bv, s5)
                hc2 = pltpu.async_copy(ones_v, a_dst.at[iv], sb, add=True)
                hir.wait()
                hgr = pltpu.async_copy(rel_hbm.at[ir], br, s6)
                hc3 = pltpu.async_copy(ones_v, a_rel.at[ir], sc, add=True)
                hgu.wait()
                hwu = pltpu.async_copy(bu, u_hbm.at[pl.ds(row0, CH)], s7)
                hgv.wait()
                hwv = pltpu.async_copy(bv, v_hbm.at[pl.ds(row0, CH)], s8)
                hgr.wait()
                hwr = pltpu.async_copy(br, r_hbm.at[pl.ds(row0, CH)], s9)
                hwu.wait()
                hwv.wait()
                hwr.wait()
                hc1.wait()
                hc2.wait()
                hc3.wait()

            return carry

        lax.fori_loop(0, slots, body, 0)

        plsc.subcore_barrier()

        for acc, o0, o1 in ((a_src, cs0_hbm, cs1_hbm),
                            (a_dst, cd0_hbm, cd1_hbm),
                            (a_rel, cr0_hbm, cr1_hbm)):
            pltpu.sync_copy(acc.at[pl.ds(r0, GR)], srow_v)

            @pl.when(cid == 0)
            def _():
                pltpu.sync_copy(srow_v, o0.at[pl.ds(r0, GR)])

            @pl.when(cid == 1)
            def _():
                pltpu.sync_copy(srow_v, o1.at[pl.ds(r0, GR)])

            @pl.when(sid == 0)
            def _():
                pltpu.sync_copy(acc.at[pl.ds(TAIL0, TAIL)], sm_s)

                @pl.when(cid == 0)
                def _():
                    pltpu.sync_copy(sm_s, o0.at[pl.ds(TAIL0, TAIL)])

                @pl.when(cid == 1)
                def _():
                    pltpu.sync_copy(sm_s, o1.at[pl.ds(TAIL0, TAIL)])

    return k(ent_embed, rel_embed, src, dst, rel, zs, ones_in)


def _sc_scatter(fc, bc, fe, be, src, dst, rel, zc, zs, ones_in, n, n_seg):
    nch = n // CH
    slots = (nch + NS - 1) // NS
    mesh = plsc.VectorSubcoreMesh(core_axis_name="c", subcore_axis_name="s")

    @functools.partial(
        pl.kernel, mesh=mesh,
        out_type=(jax.ShapeDtypeStruct((n_seg, D), jnp.float32),
                  jax.ShapeDtypeStruct((n_seg,), jnp.float32),
                  jax.ShapeDtypeStruct((n_seg, D), jnp.float32),
                  jax.ShapeDtypeStruct((n_seg,), jnp.float32),
                  jax.ShapeDtypeStruct((n_seg,), jnp.float32)),
        scratch_types=[
            pltpu.VMEM_SHARED((n_seg, D), jnp.float32),
            pltpu.VMEM_SHARED((n_seg,), jnp.float32),
            pltpu.VMEM_SHARED((n_seg,), jnp.float32),
            pltpu.VMEM((CH,), jnp.int32),
            pltpu.VMEM((CH, D), jnp.float32),
            pltpu.VMEM((CH,), jnp.float32),
            pltpu.VMEM((CH,), jnp.float32),
            pltpu.VMEM((WB, D), jnp.float32),
            pltpu.VMEM((GR,), jnp.float32),
            pltpu.VMEM((16, D), jnp.float32),
            pltpu.VMEM((16,), jnp.float32),
        ] + [pltpu.SemaphoreType.DMA] * 4,
    )
    def k(fc_hbm, bc_hbm, fe_hbm, be_hbm, src_hbm, dst_hbm, rel_hbm,
          zc_hbm, zs_hbm, ones_hbm,
          entc_hbm, ents_hbm, relc_hbm, rele_hbm, rcnt_hbm,
          acc_c, acc_s, acc_s2, idx_v, rows_v, e_v, ones_v, big_c, srow_v,
          sm_c, sm_s, s1, s2, s3, s4):
        cid = lax.axis_index("c")
        sid = lax.axis_index("s")
        r0 = sid * GR

        pltpu.sync_copy(zc_hbm, big_c)
        for j in range(GR // WB):
            pltpu.sync_copy(big_c, acc_c.at[pl.ds(r0 + j * WB, WB)])
        pltpu.sync_copy(zs_hbm, srow_v)
        pltpu.sync_copy(srow_v, acc_s.at[pl.ds(r0, GR)])
        pltpu.sync_copy(srow_v, acc_s2.at[pl.ds(r0, GR)])
        pltpu.sync_copy(ones_hbm, ones_v)

        @pl.when(sid == 0)
        def _():
            pltpu.sync_copy(zc_hbm.at[pl.ds(0, TAIL)], sm_c)
            pltpu.sync_copy(sm_c, acc_c.at[pl.ds(TAIL0, TAIL)])
            pltpu.sync_copy(zs_hbm.at[pl.ds(0, TAIL)], sm_s)
            pltpu.sync_copy(sm_s, acc_s.at[pl.ds(TAIL0, TAIL)])
            pltpu.sync_copy(sm_s, acc_s2.at[pl.ds(TAIL0, TAIL)])

        plsc.subcore_barrier()

        def scan_ent(data_c, data_e, ids):
            def body(i, carry):
                c = i * NS + sid

                @pl.when(c < nch)
                def _():
                    row0 = c * CH
                    h1 = pltpu.async_copy(ids.at[pl.ds(row0, CH)], idx_v, s1)
                    h2 = pltpu.async_copy(data_c.at[pl.ds(row0, CH)], rows_v,
                                          s2)
                    h3 = pltpu.async_copy(data_e.at[pl.ds(row0, CH)], e_v, s3)
                    h1.wait()
                    h2.wait()
                    h3.wait()
                    g1 = pltpu.async_copy(rows_v, acc_c.at[idx_v], s2,
                                          add=True)
                    g2 = pltpu.async_copy(e_v, acc_s.at[idx_v], s3, add=True)
                    g1.wait()
                    g2.wait()

                return carry
            lax.fori_loop(0, slots, body, 0)

        def scan_rel(data_c, data_e, ids):
            def body(i, carry):
                c = i * NS + sid

                @pl.when(c < nch)
                def _():
                    row0 = c * CH
                    h1 = pltpu.async_copy(ids.at[pl.ds(row0, CH)], idx_v, s1)
                    h2 = pltpu.async_copy(data_c.at[pl.ds(row0, CH)], rows_v,
                                          s2)
                    h3 = pltpu.async_copy(data_e.at[pl.ds(row0, CH)], e_v, s3)
                    h1.wait()
                    h2.wait()
                    h3.wait()
                    g1 = pltpu.async_copy(rows_v, acc_c.at[idx_v], s2,
                                          add=True)
                    g2 = pltpu.async_copy(e_v, acc_s.at[idx_v], s3, add=True)
                    g3 = pltpu.async_copy(ones_v, acc_s2.at[idx_v], s4,
                                          add=True)
                    g1.wait()
                    g2.wait()
                    g3.wait()

                return carry
            lax.fori_loop(0, slots, body, 0)

        @pl.when(cid == 0)
        def _():
            scan_ent(fc_hbm, fe_hbm, src_hbm)
            scan_ent(bc_hbm, be_hbm, dst_hbm)

        @pl.when(cid == 1)
        def _():
            scan_rel(fc_hbm, fe_hbm, rel_hbm)

        plsc.subcore_barrier()

        for j in range(GR // WB):
            rj = r0 + j * WB
            pltpu.sync_copy(acc_c.at[pl.ds(rj, WB)], big_c)

            @pl.when(cid == 0)
            def _():
                pltpu.sync_copy(big_c, entc_hbm.at[pl.ds(rj, WB)])

            @pl.when(cid == 1)
            def _():
                pltpu.sync_copy(big_c, relc_hbm.at[pl.ds(rj, WB)])

        pltpu.sync_copy(acc_s.at[pl.ds(r0, GR)], srow_v)

        @pl.when(cid == 0)
        def _():
            pltpu.sync_copy(srow_v, ents_hbm.at[pl.ds(r0, GR)])

        @pl.when(cid == 1)
        def _():
            pltpu.sync_copy(srow_v, rele_hbm.at[pl.ds(r0, GR)])
            pltpu.sync_copy(acc_s2.at[pl.ds(r0, GR)], srow_v)
            pltpu.sync_copy(srow_v, rcnt_hbm.at[pl.ds(r0, GR)])

        @pl.when(sid == 0)
        def _():
            pltpu.sync_copy(acc_c.at[pl.ds(TAIL0, TAIL)], sm_c)
            pltpu.sync_copy(acc_s.at[pl.ds(TAIL0, TAIL)], sm_s)

            @pl.when(cid == 0)
            def _():
                pltpu.sync_copy(sm_c, entc_hbm.at[pl.ds(TAIL0, TAIL)])
                pltpu.sync_copy(sm_s, ents_hbm.at[pl.ds(TAIL0, TAIL)])

            @pl.when(cid == 1)
            def _():
                pltpu.sync_copy(sm_c, relc_hbm.at[pl.ds(TAIL0, TAIL)])
                pltpu.sync_copy(sm_s, rele_hbm.at[pl.ds(TAIL0, TAIL)])
                pltpu.sync_copy(acc_s2.at[pl.ds(TAIL0, TAIL)], sm_s)
                pltpu.sync_copy(sm_s, rcnt_hbm.at[pl.ds(TAIL0, TAIL)])

    return k(fc, bc, fe, be, src, dst, rel, zc, zs, ones_in)


def _tc_stats(ent_embed, rel_embed, cs0, cs1, cd0, cd1, cr0, cr1):
    n_seg = ent_embed.shape[0]
    grid = n_seg // BK

    def body(e_ref, r_ref, u0_ref, u1_ref, d0_ref, d1_ref, r0_ref, r1_ref,
             o_ref):
        @pl.when(pl.program_id(0) == 0)
        def _():
            o_ref[...] = jnp.zeros_like(o_ref)
        ee = e_ref[...]
        rr = r_ref[...]
        wuu = u0_ref[...] + u1_ref[...] + d0_ref[...] + d1_ref[...]
        wrr = r0_ref[...] + r1_ref[...]
        we = wuu * ee
        wr2 = wrr * rr
        o_ref[pl.ds(0, 1), :] += jnp.sum(we, 0, keepdims=True)
        o_ref[pl.ds(1, 1), :] += jnp.sum(we * ee, 0, keepdims=True)
        o_ref[pl.ds(2, 1), :] += jnp.sum(wr2 * rr, 0, keepdims=True)

    blk = pl.BlockSpec((BK, D), lambda i: (i, 0))
    cblk = pl.BlockSpec((BK, 1), lambda i: (i, 0))
    return pl.pallas_call(
        body,
        grid=(grid,),
        in_specs=[blk, blk, cblk, cblk, cblk, cblk, cblk, cblk],
        out_specs=pl.BlockSpec((8, D), lambda i: (0, 0)),
        out_shape=jax.ShapeDtypeStruct((8, D), jnp.float32),
    )(ent_embed, rel_embed, cs0, cs1, cd0, cd1, cr0, cr1)


def _tc_matmul(u, v, r, w0, w1, w2, bp):
    n = u.shape[0]
    grid = n // BK

    def body(u_ref, v_ref, r_ref, w0_ref, w1_ref, w2_ref, bp_ref,
             cf_ref, cb_ref, st_ref):
        uu = u_ref[...]
        vv = v_ref[...]
        rr = r_ref[...]
        w0m = w0_ref[...]
        w1m = w1_ref[...]
        s = jnp.dot(rr, w2_ref[...], preferred_element_type=jnp.float32)
        cf = (jnp.dot(uu, w0m, preferred_element_type=jnp.float32)
              + jnp.dot(vv, w1m, preferred_element_type=jnp.float32)
              + s + bp_ref[...])
        cb = (jnp.dot(vv, w0m, preferred_element_type=jnp.float32)
              + jnp.dot(uu, w1m, preferred_element_type=jnp.float32)
              - s + bp_ref[...])
        cf_ref[...] = cf
        cb_ref[...] = cb

        @pl.when(pl.program_id(0) == 0)
        def _():
            st_ref[...] = jnp.zeros_like(st_ref)
        st_ref[pl.ds(0, 1), :] += (jnp.sum(cf, 0, keepdims=True)
                                   + jnp.sum(cb, 0, keepdims=True))
        st_ref[pl.ds(1, 1), :] += (jnp.sum(cf * cf, 0, keepdims=True)
                                   + jnp.sum(cb * cb, 0, keepdims=True))

    blk = pl.BlockSpec((BK, D), lambda i: (i, 0))
    wblk = pl.BlockSpec((D, D), lambda i: (0, 0))
    return pl.pallas_call(
        body,
        grid=(grid,),
        in_specs=[blk, blk, blk, wblk, wblk, wblk,
                  pl.BlockSpec((1, D), lambda i: (0, 0))],
        out_specs=[blk, blk, pl.BlockSpec((8, D), lambda i: (0, 0))],
        out_shape=[jax.ShapeDtypeStruct((n, D), jnp.float32),
                   jax.ShapeDtypeStruct((n, D), jnp.float32),
                   jax.ShapeDtypeStruct((8, D), jnp.float32)],
    )(u, v, r, w0, w1, w2, bp)


def _tc_edge(cf, cb, w2col, par):
    n = cf.shape[0]
    grid = n // BK

    def half(c_ref, w2_ref, par_ref, oc_ref, oe_ref):
        c = c_ref[...]
        b2 = par_ref[3, 0]
        a = jnp.dot(c, w2_ref[...], preferred_element_type=jnp.float32) + b2
        e = jnp.exp(-jnp.where(a > 0, a, 0.01 * a))
        oc_ref[...] = e * c
        oe_ref[...] = e

    def body(cf_ref, cb_ref, w2_ref, par_ref, fc_ref, bc_ref, fe_ref, be_ref):
        half(cf_ref, w2_ref, par_ref, fc_ref, fe_ref)
        half(cb_ref, w2_ref, par_ref, bc_ref, be_ref)

    blk = pl.BlockSpec((BK, D), lambda i: (i, 0))
    eblk = pl.BlockSpec((BK, 1), lambda i: (i, 0))
    return pl.pallas_call(
        body,
        grid=(grid,),
        in_specs=[blk, blk, pl.BlockSpec((D, 1), lambda i: (0, 0)),
                  pl.BlockSpec((8, D), lambda i: (0, 0))],
        out_specs=[blk, blk, eblk, eblk],
        out_shape=[jax.ShapeDtypeStruct((n, D), jnp.float32),
                   jax.ShapeDtypeStruct((n, D), jnp.float32),
                   jax.ShapeDtypeStruct((n, 1), jnp.float32),
                   jax.ShapeDtypeStruct((n, 1), jnp.float32)],
    )(cf, cb, w2col, par)


def _tc_divide(entc, ents, relc, rele, rcnt, par):
    n_seg = entc.shape[0]
    grid = n_seg // BK

    def body(ec_ref, ee_ref, rc_ref, re_ref, rn_ref, par_ref,
             he_ref, hr_ref):
        alpha = par_ref[pl.ds(0, 1), :]
        delta = par_ref[pl.ds(1, 1), :]
        ee = ee_ref[...]
        ebs = jnp.where(ee == 0.0, 1e-12, ee)
        he_ref[...] = jnp.where(ee == 0.0, 0.0,
                                alpha * (ec_ref[...] / ebs) + delta)
        cnt = jnp.maximum(rn_ref[...], 1.0)
        hr_ref[...] = (alpha * (rc_ref[...] / cnt)
                       + delta * (re_ref[...] / cnt))

    blk = pl.BlockSpec((BK, D), lambda i: (i, 0))
    eblk = pl.BlockSpec((BK, 1), lambda i: (i, 0))
    return pl.pallas_call(
        body,
        grid=(grid,),
        in_specs=[blk, eblk, blk, eblk, eblk,
                  pl.BlockSpec((8, D), lambda i: (0, 0))],
        out_specs=[blk, blk],
        out_shape=[jax.ShapeDtypeStruct((n_seg, D), jnp.float32),
                   jax.ShapeDtypeStruct((n_seg, D), jnp.float32)],
    )(entc, ents, relc, rele, rcnt, par)


@jax.jit
def kernel(triplets, ent_embed, rel_embed, W_a, b_a, W_a2, b_a2,
           gamma0, beta0, gamma1, beta1):
    n = triplets.shape[0]
    n_seg = ent_embed.shape[0]
    N = jnp.float32(2 * n)
    eps = jnp.float32(1e-5)

    src = triplets[:, 0]
    dst = triplets[:, 1]
    rel = triplets[:, 2]

    zs = jnp.zeros((GR,), jnp.float32)
    ones_in = jnp.ones((CH,), jnp.float32)

    (u, v, r, cs0, cs1, cd0, cd1, cr0, cr1) = _sc_gather(
        ent_embed, rel_embed, src, dst, rel, zs, ones_in, n, n_seg)

    st = _tc_stats(ent_embed, rel_embed,
                   cs0.reshape(n_seg, 1), cs1.reshape(n_seg, 1),
                   cd0.reshape(n_seg, 1), cd1.reshape(n_seg, 1),
                   cr0.reshape(n_seg, 1), cr1.reshape(n_seg, 1))
    s01 = st[0]
    q01 = st[1]
    m01 = s01 / N
    var01 = q01 / N - m01 * m01
    var2 = 2.0 * st[2] / N
    m = jnp.concatenate([m01, m01, jnp.zeros_like(m01)])
    var = jnp.concatenate([var01, var01, var2])
    sfold = gamma0 * jax.lax.rsqrt(var + eps)
    Wp = W_a * sfold[None, :]
    bp = (b_a + W_a @ (beta0 - m * sfold)).reshape(1, D)
    w0 = Wp[:, 0:D].T
    w1 = Wp[:, D:2 * D].T
    w2 = Wp[:, 2 * D:3 * D].T

    cf, cb, cst = _tc_matmul(u, v, r, w0, w1, w2, bp)
    m2 = cst[0] / N
    v2 = cst[1] / N - m2 * m2
    alpha = gamma1 * jax.lax.rsqrt(v2 + eps)
    delta = beta1 - m2 * alpha
    w2v = W_a2[0] * alpha
    b2 = b_a2[0] + W_a2[0] @ delta
    par = jnp.zeros((8, D), jnp.float32)
    par = par.at[0].set(alpha).at[1].set(delta)
    par = par.at[3, 0].set(b2)
    w2col = w2v.reshape(D, 1)

    fc, bc, fe, be = _tc_edge(cf, cb, w2col, par)
    fe1 = fe.reshape(n)
    be1 = be.reshape(n)

    zc = jnp.zeros((WB, D), jnp.float32)
    entc, ents, relc, rele, rcnt = _sc_scatter(
        fc, bc, fe1, be1, src, dst, rel, zc, zs, ones_in, n, n_seg)

    h_ent, h_rel = _tc_divide(entc, ents.reshape(n_seg, 1),
                              relc, rele.reshape(n_seg, 1),
                              rcnt.reshape(n_seg, 1), par)
    return (h_ent, h_rel)
